# baseline (device time: 18563 ns/iter reference)
import jax
import jax.numpy as jnp
from jax import lax
from jax.experimental import pallas as pl
from jax.experimental.pallas import tpu as pltpu

N_DEV = 4
N_LAYERS = 3
B = 128
D = 128
ROWS = B // N_DEV


def kernel(x, Win0, Wout0, Win1, Wout1, Win2, Wout2):
    def body(
        x_ref,
        win0_ref,
        wout0_ref,
        win1_ref,
        wout1_ref,
        win2_ref,
        wout2_ref,
        out_ref,
        stage_ref,
        comm_ref,
        send_sems,
        recv_sems,
    ):
        my = lax.axis_index("i")

        barrier_sem = pltpu.get_barrier_semaphore()
        for k in range(1, N_DEV):
            pl.semaphore_signal(
                barrier_sem,
                inc=1,
                device_id=((my + k) % N_DEV,),
                device_id_type=pl.DeviceIdType.MESH,
            )

        def layer(xin, win_b, wout_b):
            h = jnp.dot(xin, win_b, preferred_element_type=jnp.float32)
            h = jnp.maximum(h, 0.0)
            return jnp.dot(
                h.astype(jnp.bfloat16), wout_b, preferred_element_type=jnp.float32
            )

        win_refs = [win0_ref, win1_ref, win2_ref]
        wout_refs = [wout0_ref, wout1_ref, wout2_ref]

        xin = x_ref[:, :].astype(jnp.bfloat16)
        win_b = win_refs[0][:, :].astype(jnp.bfloat16)
        wout_b = wout_refs[0][:, :].astype(jnp.bfloat16)
        for r in range(N_LAYERS):
            partial_b = layer(xin, win_b, wout_b).astype(jnp.bfloat16)
            stage_ref[r] = partial_b
            if r == 0:
                pl.semaphore_wait(barrier_sem, N_DEV - 1)
            rdmas = []
            for k in (2, 1, 3):
                rdma = pltpu.make_async_remote_copy(
                    src_ref=stage_ref.at[r],
                    dst_ref=comm_ref.at[r, k - 1],
                    send_sem=send_sems.at[r, k - 1],
                    recv_sem=recv_sems.at[r, k - 1],
                    device_id=((my + k) % N_DEV,),
                    device_id_type=pl.DeviceIdType.MESH,
                )
                rdma.start()
                rdmas.append(rdma)
            if r < N_LAYERS - 1:
                win_b = win_refs[r + 1][:, :].astype(jnp.bfloat16)
                wout_b = wout_refs[r + 1][:, :].astype(jnp.bfloat16)
            for rdma in rdmas:
                rdma.wait_recv()
            if r < N_LAYERS - 1:
                xin = partial_b + comm_ref[r, 0] + comm_ref[r, 1] + comm_ref[r, 2]
            else:
                sl = pl.ds(my * ROWS, ROWS)
                out_ref[:, :] = (
                    stage_ref[r, sl, :].astype(jnp.float32)
                    + comm_ref[r, 0, sl, :].astype(jnp.float32)
                    + comm_ref[r, 1, sl, :].astype(jnp.float32)
                    + comm_ref[r, 2, sl, :].astype(jnp.float32)
                )
            for rdma in rdmas:
                rdma.wait_send()

    return pl.pallas_call(
        body,
        out_shape=jax.ShapeDtypeStruct((ROWS, D), jnp.float32),
        in_specs=[pl.BlockSpec(memory_space=pltpu.VMEM)] * 7,
        out_specs=pl.BlockSpec(memory_space=pltpu.VMEM),
        scratch_shapes=[
            pltpu.VMEM((N_LAYERS, B, D), jnp.bfloat16),
            pltpu.VMEM((N_LAYERS, N_DEV - 1, B, D), jnp.bfloat16),
            pltpu.SemaphoreType.DMA((N_LAYERS, N_DEV - 1)),
            pltpu.SemaphoreType.DMA((N_LAYERS, N_DEV - 1)),
        ],
        compiler_params=pltpu.CompilerParams(collective_id=0),
    )(x, Win0, Wout0, Win1, Wout1, Win2, Wout2)
